# Initial kernel scaffold; baseline (speedup 1.0000x reference)
import jax
import jax.numpy as jnp
from jax.experimental import pallas as pl


def kernel(in_feat, edge_index, W_self1, W_neigh1, b1, W_self2, W_neigh2, b2):
    raise NotImplementedError("write your pallas kernel here")


# speedup vs baseline: 3.5512x; 3.5512x over previous
import functools

import jax
import jax.numpy as jnp
from jax import lax
from jax.experimental import pallas as pl
from jax.experimental.pallas import tpu as pltpu
from jax.experimental.pallas import tpu_sc as plsc

N_NODES = 10000
N_EDGES = 320000
D = 128

NC = 2
NS = 16
NW = NC * NS

NPAD = 10240
BATCH = 128
EPW = 10240
NB = EPW // BATCH
EPAD = NW * EPW
PAD_ROW = NPAD - 1
STRIPE = NPAD // NS
HALF = STRIPE // 2

BR = 1024
GRID = NPAD // BR


def _agg_body(compute_deg, *refs):
    if compute_deg:
        (m_hbm, src_hbm, dst_hbm, z2_hbm, z1_hbm, ones_hbm,
         acc_out, deg_out,
         idx_s, idx_d, rows0, ones_v,
         acc_sh, deg_sh, sem0) = refs
    else:
        (m_hbm, src_hbm, dst_hbm, z2_hbm,
         acc_out,
         idx_s, idx_d, rows0,
         acc_sh, sem0) = refs

    c = lax.axis_index("c")
    s = lax.axis_index("s")
    wid = c * NS + s

    pltpu.sync_copy(z2_hbm, acc_sh.at[pl.ds(s * STRIPE, STRIPE)])
    pltpu.sync_copy(src_hbm.at[wid], idx_s)
    pltpu.sync_copy(dst_hbm.at[wid], idx_d)
    if compute_deg:
        pltpu.sync_copy(z1_hbm, deg_sh.at[pl.ds(s * STRIPE, STRIPE)])
        pltpu.sync_copy(ones_hbm, ones_v)
    plsc.subcore_barrier()

    def step(j):
        pltpu.sync_copy(m_hbm.at[idx_s.at[j]], rows0)
        pltpu.sync_copy(rows0, acc_sh.at[idx_d.at[j]], add=True)
        if compute_deg:
            pltpu.sync_copy(ones_v, deg_sh.at[idx_d.at[j]], add=True)

    pl.loop(0, NB)(step)
    plsc.subcore_barrier()

    pltpu.sync_copy(acc_sh.at[pl.ds(s * STRIPE, STRIPE)],
                    acc_out.at[c].at[pl.ds(s * STRIPE, STRIPE)])
    if compute_deg:
        pltpu.sync_copy(deg_sh.at[pl.ds(s * STRIPE, STRIPE)],
                        deg_out.at[c].at[pl.ds(s * STRIPE, STRIPE)])


def _make_agg(compute_deg):
    mesh = plsc.VectorSubcoreMesh(core_axis_name="c", subcore_axis_name="s")
    out_type = [jax.ShapeDtypeStruct((NC, NPAD, D), jnp.float32)]
    scratch = [
        pltpu.VMEM((NB, BATCH), jnp.int32),
        pltpu.VMEM((NB, BATCH), jnp.int32),
        pltpu.VMEM((BATCH, D), jnp.float32),
    ]
    if compute_deg:
        out_type.append(jax.ShapeDtypeStruct((NC, NPAD), jnp.float32))
        scratch += [
            pltpu.VMEM((BATCH,), jnp.float32),
        ]
    scratch.append(pltpu.VMEM_SHARED((NPAD, D), jnp.float32))
    if compute_deg:
        scratch.append(pltpu.VMEM_SHARED((NPAD,), jnp.float32))
    scratch.append(pltpu.SemaphoreType.DMA)
    return pl.kernel(
        functools.partial(_agg_body, compute_deg),
        out_type=out_type,
        mesh=mesh,
        scratch_types=scratch,
    )


def _mm2_body(x_ref, ws_ref, wn_ref, b_ref, s_ref, m_ref):
    x = x_ref[...]
    s_ref[...] = jnp.dot(x, ws_ref[...],
                         preferred_element_type=jnp.float32,
                         precision=lax.Precision.HIGHEST) + b_ref[...]
    m_ref[...] = jnp.dot(x, wn_ref[...],
                         preferred_element_type=jnp.float32,
                         precision=lax.Precision.HIGHEST)


def _combine_mm2_body(s1_ref, acc_ref, deg_ref, ws_ref, wn_ref, b_ref,
                      s2_ref, m2_ref):
    deg = deg_ref[0] + deg_ref[1]
    inv = 1.0 / jnp.maximum(deg, 1.0)
    h = s1_ref[...] + (acc_ref[0] + acc_ref[1]) * inv
    h = jnp.maximum(h, 0.0)
    s2_ref[...] = jnp.dot(h, ws_ref[...],
                          preferred_element_type=jnp.float32,
                          precision=lax.Precision.HIGHEST) + b_ref[...]
    m2_ref[...] = jnp.dot(h, wn_ref[...],
                          preferred_element_type=jnp.float32,
                          precision=lax.Precision.HIGHEST)


def _combine_body(s2_ref, acc_ref, deg_ref, out_ref):
    deg = deg_ref[0] + deg_ref[1]
    inv = 1.0 / jnp.maximum(deg, 1.0)
    out_ref[...] = s2_ref[...] + (acc_ref[0] + acc_ref[1]) * inv


_row_spec = pl.BlockSpec((BR, D), lambda i: (i, 0))
_acc_spec = pl.BlockSpec((NC, BR, D), lambda i: (0, i, 0))
_deg_spec = pl.BlockSpec((NC, BR, 1), lambda i: (0, i, 0))
_w_spec = pl.BlockSpec((D, D), lambda i: (0, 0))
_b_spec = pl.BlockSpec((1, D), lambda i: (0, 0))

_mm2 = pl.pallas_call(
    _mm2_body,
    grid=(GRID,),
    in_specs=[_row_spec, _w_spec, _w_spec, _b_spec],
    out_specs=[_row_spec, _row_spec],
    out_shape=[jax.ShapeDtypeStruct((NPAD, D), jnp.float32)] * 2,
)

_combine_mm2 = pl.pallas_call(
    _combine_mm2_body,
    grid=(GRID,),
    in_specs=[_row_spec, _acc_spec, _deg_spec, _w_spec, _w_spec, _b_spec],
    out_specs=[_row_spec, _row_spec],
    out_shape=[jax.ShapeDtypeStruct((NPAD, D), jnp.float32)] * 2,
)

_combine = pl.pallas_call(
    _combine_body,
    grid=(GRID,),
    in_specs=[_row_spec, _acc_spec, _deg_spec],
    out_specs=_row_spec,
    out_shape=jax.ShapeDtypeStruct((NPAD, D), jnp.float32),
)

_agg_deg = _make_agg(True)


def kernel(in_feat, edge_index, W_self1, W_neigh1, b1, W_self2, W_neigh2, b2):
    x = jnp.pad(in_feat, ((0, NPAD - N_NODES), (0, 0)))
    src = edge_index[0].astype(jnp.int32)
    dst = edge_index[1].astype(jnp.int32)
    src = jnp.pad(src, (0, EPAD - N_EDGES), constant_values=PAD_ROW)
    dst = jnp.pad(dst, (0, EPAD - N_EDGES), constant_values=PAD_ROW)
    src = src.reshape(NW, NB, BATCH)
    dst = dst.reshape(NW, NB, BATCH)

    z2 = jnp.zeros((STRIPE, D), jnp.float32)
    z1 = jnp.zeros((STRIPE,), jnp.float32)
    ones = jnp.ones((BATCH,), jnp.float32)
    b1r = b1.reshape(1, D)
    b2r = b2.reshape(1, D)

    s1, m1 = _mm2(x, W_self1, W_neigh1, b1r)
    acc1, deg = _agg_deg(m1, src, dst, z2, z1, ones)
    s2, m2 = _combine_mm2(s1, acc1, deg.reshape(NC, NPAD, 1),
                          W_self2, W_neigh2, b2r)
    acc2, _ = _agg_deg(m2, src, dst, z2, z1, ones)
    out = _combine(s2, acc2, deg.reshape(NC, NPAD, 1))
    return out[:N_NODES]


# speedup vs baseline: 10.0067x; 2.8179x over previous
import functools

import jax
import jax.numpy as jnp
from jax import lax
from jax.experimental import pallas as pl
from jax.experimental.pallas import tpu as pltpu
from jax.experimental.pallas import tpu_sc as plsc

N_NODES = 10000
N_EDGES = 320000
D = 128

NC = 2
NS = 16
NW = NC * NS

NPAD = 10240
BATCH = 125
EPW = N_EDGES // NW
NB = EPW // BATCH
K = 16
NPHASE = NB // K
STRIPE = NPAD // NS

BR = 1024
GRID = NPAD // BR


def _agg_body(compute_deg, *refs):
    if compute_deg:
        (m_hbm, src_hbm, dst_hbm, z2_hbm, z1_hbm, ones_hbm,
         acc_out, deg_out,
         idx_s, idx_d, rows0, rows1, ones_v,
         acc_sh, deg_sh, sem0, sem1) = refs
    else:
        (m_hbm, src_hbm, dst_hbm, z2_hbm,
         acc_out,
         idx_s, idx_d, rows0, rows1,
         acc_sh, sem0, sem1) = refs

    c = lax.axis_index("c")
    s = lax.axis_index("s")
    wid = c * NS + s
    rows = (rows0, rows1)
    sems = (sem0, sem1)

    pltpu.sync_copy(z2_hbm, acc_sh.at[pl.ds(s * STRIPE, STRIPE)])
    if compute_deg:
        pltpu.sync_copy(z1_hbm, deg_sh.at[pl.ds(s * STRIPE, STRIPE)])
        pltpu.sync_copy(ones_hbm, ones_v)
    plsc.subcore_barrier()

    for p in range(NPHASE):
        pltpu.sync_copy(src_hbm.at[wid].at[pl.ds(p * K, K)], idx_s)
        pltpu.sync_copy(dst_hbm.at[wid].at[pl.ds(p * K, K)], idx_d)
        pltpu.async_copy(m_hbm.at[idx_s.at[0]], rows0, sem0)

        def step(j):
            for b in range(2):
                cur = j + b
                nxt = cur + 1
                pltpu.make_async_copy(m_hbm.at[idx_s.at[cur]],
                                      rows[b], sems[b]).wait()

                @pl.when(nxt < K)
                def _():
                    pltpu.async_copy(m_hbm.at[idx_s.at[nxt]],
                                     rows[1 - b], sems[1 - b])

                pltpu.sync_copy(rows[b], acc_sh.at[idx_d.at[cur]], add=True)
                if compute_deg:
                    pltpu.sync_copy(ones_v, deg_sh.at[idx_d.at[cur]],
                                    add=True)

        pl.loop(0, K, step=2)(step)
    plsc.subcore_barrier()

    pltpu.sync_copy(acc_sh.at[pl.ds(s * STRIPE, STRIPE)],
                    acc_out.at[c].at[pl.ds(s * STRIPE, STRIPE)])
    if compute_deg:
        pltpu.sync_copy(deg_sh.at[pl.ds(s * STRIPE, STRIPE)],
                        deg_out.at[c].at[pl.ds(s * STRIPE, STRIPE)])


def _make_agg(compute_deg):
    mesh = plsc.VectorSubcoreMesh(core_axis_name="c", subcore_axis_name="s")
    acc_t = jax.ShapeDtypeStruct((NC, NPAD, D), jnp.float32)
    out_type = [acc_t] if compute_deg else acc_t
    scratch = [
        pltpu.VMEM((K, BATCH), jnp.int32),
        pltpu.VMEM((K, BATCH), jnp.int32),
        pltpu.VMEM((BATCH, D), jnp.float32),
        pltpu.VMEM((BATCH, D), jnp.float32),
    ]
    if compute_deg:
        out_type.append(jax.ShapeDtypeStruct((NC, NPAD), jnp.float32))
        scratch += [
            pltpu.VMEM((BATCH,), jnp.float32),
        ]
    scratch.append(pltpu.VMEM_SHARED((NPAD, D), jnp.float32))
    if compute_deg:
        scratch.append(pltpu.VMEM_SHARED((NPAD,), jnp.float32))
    scratch += [pltpu.SemaphoreType.DMA, pltpu.SemaphoreType.DMA]
    return pl.kernel(
        functools.partial(_agg_body, compute_deg),
        out_type=out_type,
        mesh=mesh,
        scratch_types=scratch,
    )


def _mm2_body(x_ref, ws_ref, wn_ref, b_ref, s_ref, m_ref):
    x = x_ref[...]
    s_ref[...] = jnp.dot(x, ws_ref[...],
                         preferred_element_type=jnp.float32,
                         precision=lax.Precision.HIGHEST) + b_ref[...]
    m_ref[...] = jnp.dot(x, wn_ref[...],
                         preferred_element_type=jnp.float32,
                         precision=lax.Precision.HIGHEST)


def _combine_mm2_body(s1_ref, acc_ref, deg_ref, ws_ref, wn_ref, b_ref,
                      s2_ref, m2_ref):
    deg = deg_ref[0] + deg_ref[1]
    inv = 1.0 / jnp.maximum(deg, 1.0)
    h = s1_ref[...] + (acc_ref[0] + acc_ref[1]) * inv
    h = jnp.maximum(h, 0.0)
    s2_ref[...] = jnp.dot(h, ws_ref[...],
                          preferred_element_type=jnp.float32,
                          precision=lax.Precision.HIGHEST) + b_ref[...]
    m2_ref[...] = jnp.dot(h, wn_ref[...],
                          preferred_element_type=jnp.float32,
                          precision=lax.Precision.HIGHEST)


def _combine_body(s2_ref, acc_ref, deg_ref, out_ref):
    deg = deg_ref[0] + deg_ref[1]
    inv = 1.0 / jnp.maximum(deg, 1.0)
    out_ref[...] = s2_ref[...] + (acc_ref[0] + acc_ref[1]) * inv


_row_spec = pl.BlockSpec((BR, D), lambda i: (i, 0))
_acc_spec = pl.BlockSpec((NC, BR, D), lambda i: (0, i, 0))
_deg_spec = pl.BlockSpec((NC, BR, 1), lambda i: (0, i, 0))
_w_spec = pl.BlockSpec((D, D), lambda i: (0, 0))
_b_spec = pl.BlockSpec((1, D), lambda i: (0, 0))

_mm2 = pl.pallas_call(
    _mm2_body,
    grid=(GRID,),
    in_specs=[_row_spec, _w_spec, _w_spec, _b_spec],
    out_specs=[_row_spec, _row_spec],
    out_shape=[jax.ShapeDtypeStruct((NPAD, D), jnp.float32)] * 2,
)

_combine_mm2 = pl.pallas_call(
    _combine_mm2_body,
    grid=(GRID,),
    in_specs=[_row_spec, _acc_spec, _deg_spec, _w_spec, _w_spec, _b_spec],
    out_specs=[_row_spec, _row_spec],
    out_shape=[jax.ShapeDtypeStruct((NPAD, D), jnp.float32)] * 2,
)

_combine = pl.pallas_call(
    _combine_body,
    grid=(GRID,),
    in_specs=[_row_spec, _acc_spec, _deg_spec],
    out_specs=_row_spec,
    out_shape=jax.ShapeDtypeStruct((NPAD, D), jnp.float32),
)

_agg_deg = _make_agg(True)
_agg_nodeg = _make_agg(False)


def kernel(in_feat, edge_index, W_self1, W_neigh1, b1, W_self2, W_neigh2, b2):
    x = jnp.pad(in_feat, ((0, NPAD - N_NODES), (0, 0)))
    src = edge_index[0].astype(jnp.int32).reshape(NW, NB, BATCH)
    dst = edge_index[1].astype(jnp.int32).reshape(NW, NB, BATCH)

    z2 = jnp.zeros((STRIPE, D), jnp.float32)
    z1 = jnp.zeros((STRIPE,), jnp.float32)
    ones = jnp.ones((BATCH,), jnp.float32)
    b1r = b1.reshape(1, D)
    b2r = b2.reshape(1, D)

    s1, m1 = _mm2(x, W_self1, W_neigh1, b1r)
    acc1, deg = _agg_deg(m1, src, dst, z2, z1, ones)
    s2, m2 = _combine_mm2(s1, acc1, deg.reshape(NC, NPAD, 1),
                          W_self2, W_neigh2, b2r)
    acc2 = _agg_nodeg(m2, src, dst, z2)
    out = _combine(s2, acc2, deg.reshape(NC, NPAD, 1))
    return out[:N_NODES]


# speedup vs baseline: 10.5185x; 1.0511x over previous
import functools

import jax
import jax.numpy as jnp
from jax import lax
from jax.experimental import pallas as pl
from jax.experimental.pallas import tpu as pltpu
from jax.experimental.pallas import tpu_sc as plsc

N_NODES = 10000
N_EDGES = 320000
D = 128

NC = 2
NS = 16
NW = NC * NS

NPAD = 10240
BATCH = 125
EPW = N_EDGES // NW
NB = EPW // BATCH
K = 16
NPHASE = NB // K
STRIPE = NPAD // NS

BR = 1024
GRID = NPAD // BR


def _agg_body(compute_deg, *refs):
    if compute_deg:
        (m_hbm, src_hbm, dst_hbm, z2_hbm, z1_hbm, ones_hbm,
         acc_out, deg_out,
         idx_s0, idx_s1, idx_d0, idx_d1, rows0, rows1, ones_v,
         acc_sh, deg_sh, sem0, sem1, isem0, isem1) = refs
    else:
        (m_hbm, src_hbm, dst_hbm, z2_hbm,
         acc_out,
         idx_s0, idx_s1, idx_d0, idx_d1, rows0, rows1,
         acc_sh, sem0, sem1, isem0, isem1) = refs

    c = lax.axis_index("c")
    s = lax.axis_index("s")
    wid = c * NS + s
    rows = (rows0, rows1)
    sems = (sem0, sem1)
    idx_s = (idx_s0, idx_s1)
    idx_d = (idx_d0, idx_d1)
    isems = (isem0, isem1)
    src_w = src_hbm.at[wid]
    dst_w = dst_hbm.at[wid]

    pltpu.async_copy(src_w.at[pl.ds(0, K)], idx_s0, isem0)
    pltpu.async_copy(dst_w.at[pl.ds(0, K)], idx_d0, isem0)
    pltpu.sync_copy(z2_hbm, acc_sh.at[pl.ds(s * STRIPE, STRIPE)])
    if compute_deg:
        pltpu.sync_copy(z1_hbm, deg_sh.at[pl.ds(s * STRIPE, STRIPE)])
        pltpu.sync_copy(ones_hbm, ones_v)
    plsc.subcore_barrier()

    for p in range(NPHASE):
        q = p & 1
        pltpu.make_async_copy(src_w.at[pl.ds(p * K, K)],
                              idx_s[q], isems[q]).wait()
        pltpu.make_async_copy(dst_w.at[pl.ds(p * K, K)],
                              idx_d[q], isems[q]).wait()
        if p + 1 < NPHASE:
            pltpu.async_copy(src_w.at[pl.ds((p + 1) * K, K)],
                             idx_s[1 - q], isems[1 - q])
            pltpu.async_copy(dst_w.at[pl.ds((p + 1) * K, K)],
                             idx_d[1 - q], isems[1 - q])
        pltpu.async_copy(m_hbm.at[idx_s[q].at[0]], rows0, sem0)

        def step(j):
            for b in range(2):
                cur = j + b
                nxt = cur + 1
                pltpu.make_async_copy(m_hbm.at[idx_s[q].at[cur]],
                                      rows[b], sems[b]).wait()

                @pl.when(nxt < K)
                def _():
                    pltpu.async_copy(m_hbm.at[idx_s[q].at[nxt]],
                                     rows[1 - b], sems[1 - b])

                pltpu.sync_copy(rows[b], acc_sh.at[idx_d[q].at[cur]],
                                add=True)
                if compute_deg:
                    pltpu.sync_copy(ones_v, deg_sh.at[idx_d[q].at[cur]],
                                    add=True)

        pl.loop(0, K, step=2)(step)
    plsc.subcore_barrier()

    pltpu.sync_copy(acc_sh.at[pl.ds(s * STRIPE, STRIPE)],
                    acc_out.at[c].at[pl.ds(s * STRIPE, STRIPE)])
    if compute_deg:
        pltpu.sync_copy(deg_sh.at[pl.ds(s * STRIPE, STRIPE)],
                        deg_out.at[c].at[pl.ds(s * STRIPE, STRIPE)])


def _make_agg(compute_deg):
    mesh = plsc.VectorSubcoreMesh(core_axis_name="c", subcore_axis_name="s")
    acc_t = jax.ShapeDtypeStruct((NC, NPAD, D), jnp.float32)
    out_type = [acc_t] if compute_deg else acc_t
    scratch = [
        pltpu.VMEM((K, BATCH), jnp.int32),
        pltpu.VMEM((K, BATCH), jnp.int32),
        pltpu.VMEM((K, BATCH), jnp.int32),
        pltpu.VMEM((K, BATCH), jnp.int32),
        pltpu.VMEM((BATCH, D), jnp.float32),
        pltpu.VMEM((BATCH, D), jnp.float32),
    ]
    if compute_deg:
        out_type.append(jax.ShapeDtypeStruct((NC, NPAD), jnp.float32))
        scratch += [
            pltpu.VMEM((BATCH,), jnp.float32),
        ]
    scratch.append(pltpu.VMEM_SHARED((NPAD, D), jnp.float32))
    if compute_deg:
        scratch.append(pltpu.VMEM_SHARED((NPAD,), jnp.float32))
    scratch += [pltpu.SemaphoreType.DMA, pltpu.SemaphoreType.DMA,
                pltpu.SemaphoreType.DMA, pltpu.SemaphoreType.DMA]
    return pl.kernel(
        functools.partial(_agg_body, compute_deg),
        out_type=out_type,
        mesh=mesh,
        scratch_types=scratch,
    )


def _mm_m_body(x_ref, wn_ref, m_ref):
    m_ref[...] = jnp.dot(x_ref[...], wn_ref[...],
                         preferred_element_type=jnp.float32,
                         precision=lax.Precision.HIGHEST)


def _mm_s_body(x_ref, ws_ref, b_ref, s_ref):
    s_ref[...] = jnp.dot(x_ref[...], ws_ref[...],
                         preferred_element_type=jnp.float32,
                         precision=lax.Precision.HIGHEST) + b_ref[...]


def _combine_mm_body(s1_ref, acc_ref, deg_ref, wn_ref, h_ref, m2_ref):
    deg = deg_ref[0] + deg_ref[1]
    inv = 1.0 / jnp.maximum(deg, 1.0)
    h = s1_ref[...] + (acc_ref[0] + acc_ref[1]) * inv
    h = jnp.maximum(h, 0.0)
    h_ref[...] = h
    m2_ref[...] = jnp.dot(h, wn_ref[...],
                          preferred_element_type=jnp.float32,
                          precision=lax.Precision.HIGHEST)


def _combine_body(s2_ref, acc_ref, deg_ref, out_ref):
    deg = deg_ref[0] + deg_ref[1]
    inv = 1.0 / jnp.maximum(deg, 1.0)
    out_ref[...] = s2_ref[...] + (acc_ref[0] + acc_ref[1]) * inv


_row_spec = pl.BlockSpec((BR, D), lambda i: (i, 0))
_acc_spec = pl.BlockSpec((NC, BR, D), lambda i: (0, i, 0))
_deg_spec = pl.BlockSpec((NC, BR, 1), lambda i: (0, i, 0))
_w_spec = pl.BlockSpec((D, D), lambda i: (0, 0))
_b_spec = pl.BlockSpec((1, D), lambda i: (0, 0))

_mm_m = pl.pallas_call(
    _mm_m_body,
    grid=(GRID,),
    in_specs=[_row_spec, _w_spec],
    out_specs=_row_spec,
    out_shape=jax.ShapeDtypeStruct((NPAD, D), jnp.float32),
)

_mm_s = pl.pallas_call(
    _mm_s_body,
    grid=(GRID,),
    in_specs=[_row_spec, _w_spec, _b_spec],
    out_specs=_row_spec,
    out_shape=jax.ShapeDtypeStruct((NPAD, D), jnp.float32),
)

_combine_mm = pl.pallas_call(
    _combine_mm_body,
    grid=(GRID,),
    in_specs=[_row_spec, _acc_spec, _deg_spec, _w_spec],
    out_specs=[_row_spec, _row_spec],
    out_shape=[jax.ShapeDtypeStruct((NPAD, D), jnp.float32)] * 2,
)

_combine = pl.pallas_call(
    _combine_body,
    grid=(GRID,),
    in_specs=[_row_spec, _acc_spec, _deg_spec],
    out_specs=_row_spec,
    out_shape=jax.ShapeDtypeStruct((NPAD, D), jnp.float32),
)

_agg_deg = _make_agg(True)
_agg_nodeg = _make_agg(False)


def kernel(in_feat, edge_index, W_self1, W_neigh1, b1, W_self2, W_neigh2, b2):
    x = jnp.pad(in_feat, ((0, NPAD - N_NODES), (0, 0)))
    src = edge_index[0].astype(jnp.int32).reshape(NW, NB, BATCH)
    dst = edge_index[1].astype(jnp.int32).reshape(NW, NB, BATCH)

    z2 = jnp.zeros((STRIPE, D), jnp.float32)
    z1 = jnp.zeros((STRIPE,), jnp.float32)
    ones = jnp.ones((BATCH,), jnp.float32)
    b1r = b1.reshape(1, D)
    b2r = b2.reshape(1, D)

    m1 = _mm_m(x, W_neigh1)
    acc1, deg = _agg_deg(m1, src, dst, z2, z1, ones)
    s1 = _mm_s(x, W_self1, b1r)
    h, m2 = _combine_mm(s1, acc1, deg.reshape(NC, NPAD, 1), W_neigh2)
    acc2 = _agg_nodeg(m2, src, dst, z2)
    s2 = _mm_s(h, W_self2, b2r)
    out = _combine(s2, acc2, deg.reshape(NC, NPAD, 1))
    return out[:N_NODES]
